# Initial kernel scaffold; baseline (speedup 1.0000x reference)
import functools

import jax
import jax.numpy as jnp
from jax import lax
from jax.experimental import pallas as pl
from jax.experimental.pallas import tpu as pltpu
from jax.experimental.pallas import tpu_sc as plsc

_N = 16384
_EMB = 10
_OUT = 4
_NC = 2
_NS = 16
_NW = _NC * _NS
_ROWS_PER_TILE = _N // _NS
_CHUNK_WORDS = (_N * _OUT) // _NW
_GCH = 128
_NGCH = _ROWS_PER_TILE // _GCH
_WB = _OUT * _EMB + _OUT + 4

_mesh = plsc.VectorSubcoreMesh(core_axis_name="c", subcore_axis_name="s")


@functools.partial(
    pl.kernel,
    out_type=jax.ShapeDtypeStruct((_N * _OUT,), jnp.float32),
    mesh=_mesh,
    scratch_types=[
        pltpu.VMEM((_ROWS_PER_TILE,), jnp.int32),
        pltpu.VMEM((_ROWS_PER_TILE, _EMB), jnp.float32),
        pltpu.VMEM((_GCH,), jnp.int32),
        pltpu.VMEM((_WB,), jnp.float32),
        pltpu.VMEM((16,), jnp.float32),
        pltpu.VMEM((16,), jnp.float32),
        pltpu.VMEM((_CHUNK_WORDS,), jnp.float32),
        pltpu.VMEM_SHARED((1, _EMB), jnp.float32),
    ],
)
def _sc_bag(text_hbm, wb_hbm, table_hbm, out_hbm,
            idx_v, rows_v, zidx_v, wb_v, cs_v, zrow_v, chunk_v, acc_sh):
    c = lax.axis_index("c")
    s = lax.axis_index("s")
    wid = s * _NC + c
    chunk = (_NW - 1) - wid
    owner = jnp.logical_and(c == 0, s == 0)

    pltpu.sync_copy(wb_hbm, wb_v)

    @pl.when(owner)
    def _():
        zrow_v[...] = jnp.zeros((16,), jnp.float32)
        pltpu.sync_copy(zrow_v.at[pl.ds(0, _EMB)], acc_sh.at[0])

    plsc.subcore_barrier()

    @pl.when(c == 0)
    def _():
        zv = jnp.zeros((16,), jnp.int32)
        for i in range(_GCH // 16):
            zidx_v[pl.ds(i * 16, 16)] = zv
        pltpu.sync_copy(
            text_hbm.at[pl.ds(s * _ROWS_PER_TILE, _ROWS_PER_TILE)], idx_v)
        for g in range(_NGCH):
            pltpu.sync_copy(
                table_hbm.at[idx_v.at[pl.ds(g * _GCH, _GCH)]],
                rows_v.at[pl.ds(g * _GCH, _GCH)])
        for g in range(_NGCH):
            pltpu.sync_copy(
                rows_v.at[pl.ds(g * _GCH, _GCH)], acc_sh.at[zidx_v], add=True)

    iota = lax.broadcasted_iota(jnp.int32, (16,), 0)
    bpat = plsc.load_gather(wb_v, [_OUT * _EMB + lax.rem(iota, _OUT)])
    for i in range(_CHUNK_WORDS // 16):
        chunk_v[pl.ds(i * 16, 16)] = bpat

    @pl.when(jnp.logical_not(owner))
    def _():
        pltpu.sync_copy(chunk_v,
                        out_hbm.at[pl.ds(chunk * _CHUNK_WORDS, _CHUNK_WORDS)])

    plsc.subcore_barrier()

    @pl.when(owner)
    def _():
        cs_v[...] = jnp.zeros((16,), jnp.float32)
        pltpu.sync_copy(acc_sh.at[0], cs_v.at[pl.ds(0, _EMB)])
        mean = cs_v[...] * (1.0 / _N)
        for o in range(_OUT):
            wv = wb_v[pl.ds(o * _EMB, 16)]
            y = jnp.sum(mean * wv)
            chunk_v[_CHUNK_WORDS - _OUT + o] = y + wb_v[_OUT * _EMB + o]
        pltpu.sync_copy(
            chunk_v,
            out_hbm.at[pl.ds((_NW - 1) * _CHUNK_WORDS, _CHUNK_WORDS)])


@jax.jit
def kernel(text, table, W, b):
    wb = jnp.concatenate([
        W.reshape(-1).astype(jnp.float32),
        b.reshape(-1).astype(jnp.float32),
        jnp.zeros((4,), jnp.float32),
    ])
    flat = _sc_bag(text.astype(jnp.int32), wb, table.astype(jnp.float32))
    return flat.reshape(_N, _OUT)


# speedup vs baseline: 2.2530x; 2.2530x over previous
import functools

import jax
import jax.numpy as jnp
from jax import lax
from jax.experimental import pallas as pl
from jax.experimental.pallas import tpu as pltpu
from jax.experimental.pallas import tpu_sc as plsc

_N = 16384
_EMB = 10
_OUT = 4
_NC = 2
_NS = 16
_NW = _NC * _NS
_D = 16
_R = (_N * 0 + 1000000 * _EMB) // _D
_ROWS_PER_TILE = _N // _NS
_CHUNK_WORDS = (_N * _OUT) // _NW
_GCH = 128
_NGCH = _ROWS_PER_TILE // _GCH
_WB = _OUT * _EMB + 16

_mesh = plsc.VectorSubcoreMesh(core_axis_name="c", subcore_axis_name="s")


@functools.partial(
    pl.kernel,
    out_type=jax.ShapeDtypeStruct((_N * _OUT,), jnp.float32),
    mesh=_mesh,
    compiler_params=pltpu.CompilerParams(
        needs_layout_passes=False, use_tc_tiling_on_sc=False),
    scratch_types=[
        pltpu.VMEM((_ROWS_PER_TILE,), jnp.int32),
        pltpu.VMEM((_ROWS_PER_TILE,), jnp.int32),
        pltpu.VMEM((_ROWS_PER_TILE,), jnp.int32),
        pltpu.VMEM((_NGCH, _GCH), jnp.int32),
        pltpu.VMEM((_NGCH, _GCH), jnp.int32),
        pltpu.VMEM((_ROWS_PER_TILE, _D), jnp.float32),
        pltpu.VMEM((_ROWS_PER_TILE, _D), jnp.float32),
        pltpu.VMEM((_WB,), jnp.float32),
        pltpu.VMEM((16,), jnp.float32),
        pltpu.VMEM((16 * _D,), jnp.float32),
        pltpu.VMEM((_CHUNK_WORDS,), jnp.float32),
        pltpu.VMEM_SHARED((16, _D), jnp.float32),
    ],
)
def _sc_bag(text_hbm, wb_hbm, tbl_hbm, out_hbm,
            idx_v, pa_v, pb_v, oa2_v, ob2_v,
            rowsa_v, rowsb_v, wb_v, z16_v, accflat_v, chunk_v, acc_sh):
    c = lax.axis_index("c")
    s = lax.axis_index("s")
    wid = s * _NC + c
    chunk = (_NW - 1) - wid
    owner = jnp.logical_and(c == 0, s == 0)

    pltpu.sync_copy(wb_hbm, wb_v)

    @pl.when(owner)
    def _():
        z16_v[...] = jnp.zeros((16,), jnp.float32)
        for r in range(16):
            pltpu.sync_copy(z16_v, acc_sh.at[r])

    plsc.subcore_barrier()

    @pl.when(c == 0)
    def _():
        pltpu.sync_copy(
            text_hbm.at[pl.ds(s * _ROWS_PER_TILE, _ROWS_PER_TILE)], idx_v)
        for blk in range(_ROWS_PER_TILE // 16):
            t = idx_v[pl.ds(blk * 16, 16)]
            a = t * _EMB
            p = lax.shift_right_logical(a, 4)
            o = lax.bitwise_and(a, 15)
            pa_v[pl.ds(blk * 16, 16)] = p
            pb_v[pl.ds(blk * 16, 16)] = jnp.minimum(p + 1, _R - 1)
            oa2_v[blk // 8, pl.ds((blk % 8) * 16, 16)] = o
            ob2_v[blk // 8, pl.ds((blk % 8) * 16, 16)] = o + 1
        for g in range(_NGCH):
            pltpu.sync_copy(
                tbl_hbm.at[pa_v.at[pl.ds(g * _GCH, _GCH)]],
                rowsa_v.at[pl.ds(g * _GCH, _GCH)])
            pltpu.sync_copy(
                tbl_hbm.at[pb_v.at[pl.ds(g * _GCH, _GCH)]],
                rowsb_v.at[pl.ds(g * _GCH, _GCH)])
        for g in range(_NGCH):
            pltpu.sync_copy(rowsa_v.at[pl.ds(g * _GCH, _GCH)],
                            acc_sh.at[oa2_v.at[g]], add=True)
            pltpu.sync_copy(rowsb_v.at[pl.ds(g * _GCH, _GCH)],
                            acc_sh.at[ob2_v.at[g]], add=True)

    iota = lax.broadcasted_iota(jnp.int32, (16,), 0)
    bpat = wb_v[pl.ds(_OUT * _EMB, 16)]
    for i in range(_CHUNK_WORDS // 16):
        chunk_v[pl.ds(i * 16, 16)] = bpat

    @pl.when(jnp.logical_not(owner))
    def _():
        pltpu.sync_copy(chunk_v,
                        out_hbm.at[pl.ds(chunk * _CHUNK_WORDS, _CHUNK_WORDS)])

    plsc.subcore_barrier()

    @pl.when(owner)
    def _():
        for r in range(16):
            pltpu.sync_copy(acc_sh.at[r], accflat_v.at[pl.ds(_D * r, _D)])
        cs = jnp.zeros((16,), jnp.float32)
        for o in range(0, 16, 2):
            cs = cs + accflat_v[pl.ds(17 * o, 16)]
        cs = jnp.where(iota < _EMB, cs, 0.0)
        mean = cs * (1.0 / _N)
        yv = bpat
        for o in range(_OUT):
            wv = wb_v[pl.ds(o * _EMB, 16)]
            y = jnp.sum(mean * wv) + bpat[o]
            yv = jnp.where(iota == 16 - _OUT + o, y, yv)
        chunk_v[pl.ds(_CHUNK_WORDS - 16, 16)] = yv
        pltpu.sync_copy(
            chunk_v,
            out_hbm.at[pl.ds((_NW - 1) * _CHUNK_WORDS, _CHUNK_WORDS)])


@jax.jit
def kernel(text, table, W, b):
    wb = jnp.concatenate([
        W.reshape(-1).astype(jnp.float32),
        jnp.tile(b.reshape(-1).astype(jnp.float32), 4),
    ])
    tbl = table.astype(jnp.float32).reshape(_R, _D)
    flat = _sc_bag(text.astype(jnp.int32), wb, tbl)
    return flat.reshape(_N, _OUT)
